# Initial kernel scaffold; baseline (speedup 1.0000x reference)
import jax
import jax.numpy as jnp
from jax.experimental import pallas as pl


def kernel(x, edge_index, batch, W_embed, b_embed, W_g0, b_g0, W_g1, b_g1, W_g2, b_g2, W_t1, b_t1, W_t2, b_t2):
    raise NotImplementedError("write your pallas kernel here")


# speedup vs baseline: 6.8338x; 6.8338x over previous
import functools

import jax
import jax.numpy as jnp
from jax import lax
from jax.experimental import pallas as pl
from jax.experimental.pallas import tpu as pltpu
from jax.experimental.pallas import tpu_sc as plsc

N = 10000
E = 320000
NUM_GRAPHS = 256
NPAD = 10240
EPAD = 327680
NC = 2
NS = 16
CHUNK = 128
EPT = EPAD // NS
EPW = EPAD // (NC * NS)
RPT = NPAD // NS
NSEG = 4
SEGC = EPT // CHUNK // NSEG
RBLK = 512
NBLK = NPAD // RBLK

_mesh = plsc.VectorSubcoreMesh(core_axis_name="c", subcore_axis_name="s",
                               num_cores=NC, num_subcores=NS)


def _fill2d(ref, nrows, ncols, val):
    vec = jnp.full((16,), val, jnp.float32)

    def body(r, _):
        for k in range(ncols // 16):
            ref[r, pl.ds(k * 16, 16)] = vec
        return 0

    lax.fori_loop(0, nrows, body, 0)


def _deg_body(dstd_hbm, out_hbm, idx_v, ones_v, zer_v, acc_sh, sem):
    c = lax.axis_index("c")
    s = lax.axis_index("s")
    wid = c * NS + s
    _fill2d(ones_v, CHUNK, 16, 1.0)
    _fill2d(zer_v, CHUNK, 16, 0.0)
    for k in range(RPT // CHUNK):
        pltpu.sync_copy(zer_v, acc_sh.at[pl.ds(s * RPT + k * CHUNK, CHUNK)])
    plsc.subcore_barrier()
    pltpu.sync_copy(dstd_hbm.at[wid], idx_v)

    def body(j, _):
        pltpu.sync_copy(ones_v, acc_sh.at[idx_v.at[j]], add=True)
        return 0

    lax.fori_loop(0, EPW // CHUNK, body, 0)
    plsc.subcore_barrier()
    pltpu.sync_copy(acc_sh.at[pl.ds(s * RPT, RPT)],
                    out_hbm.at[pl.ds(c * NPAD + s * RPT, RPT)])


_deg_pass = functools.partial(
    pl.kernel,
    out_type=jax.ShapeDtypeStruct((NC * NPAD, 16), jnp.float32),
    mesh=_mesh,
    scratch_types=[
        pltpu.VMEM((EPW // CHUNK, CHUNK), jnp.int32),
        pltpu.VMEM((CHUNK, 16), jnp.float32),
        pltpu.VMEM((CHUNK, 16), jnp.float32),
        pltpu.VMEM_SHARED((NPAD, 16), jnp.float32),
        pltpu.SemaphoreType.DMA,
    ],
)(_deg_body)


def _mp_body(y_hbm, src_hbm, dst_hbm, out_hbm, src_v, dst_v, rows_v, acc_sh, sem):
    c = lax.axis_index("c")
    s = lax.axis_index("s")
    wid = c * NS + s
    _fill2d(rows_v, CHUNK, 128, 0.0)
    for k in range(RPT // CHUNK):
        pltpu.sync_copy(rows_v, acc_sh.at[pl.ds(s * RPT + k * CHUNK, CHUNK)])
    plsc.subcore_barrier()

    def seg_body(g, _):
        pltpu.sync_copy(src_hbm.at[wid, g], src_v)
        pltpu.sync_copy(dst_hbm.at[s, g], dst_v)

        def body(j, _):
            pltpu.async_copy(y_hbm.at[src_v.at[j]], rows_v, sem).wait()
            pltpu.sync_copy(rows_v, acc_sh.at[dst_v.at[j]], add=True)
            return 0

        lax.fori_loop(0, SEGC, body, 0)
        return 0

    lax.fori_loop(0, NSEG, seg_body, 0)
    plsc.subcore_barrier()
    pltpu.sync_copy(acc_sh.at[pl.ds(s * RPT, RPT)],
                    out_hbm.at[pl.ds(c * NPAD + s * RPT, RPT)])


_mp_pass = functools.partial(
    pl.kernel,
    out_type=jax.ShapeDtypeStruct((NC * NPAD, 128), jnp.float32),
    mesh=_mesh,
    scratch_types=[
        pltpu.VMEM((SEGC, CHUNK), jnp.int32),
        pltpu.VMEM((SEGC, CHUNK), jnp.int32),
        pltpu.VMEM((CHUNK, 128), jnp.float32),
        pltpu.VMEM_SHARED((NPAD, 128), jnp.float32),
        pltpu.SemaphoreType.DMA,
    ],
)(_mp_body)


def _dinv_of(degp_ref):
    deg = 1.0 + jnp.sum(degp_ref[:, :, 0:1], axis=0)
    return lax.rsqrt(jnp.maximum(deg, 1.0))


def _embed_body(x_ref, we_ref, be_ref, wg_ref, degp_ref, out_ref):
    dinv = _dinv_of(degp_ref)
    h = jnp.dot(x_ref[...], we_ref[...], preferred_element_type=jnp.float32)
    h = h + be_ref[...]
    y = jnp.dot(h, wg_ref[...], preferred_element_type=jnp.float32) * dinv
    out_ref[0] = y[:, :128]
    out_ref[1] = y[:, 128:]


def _mid_body(es_ref, y_ref, degp_ref, b_ref, w_ref, out_ref):
    dinv = _dinv_of(degp_ref)
    z = jnp.concatenate([es_ref[0] + y_ref[0], es_ref[1] + y_ref[1]], axis=1)
    h = jnp.maximum(z * dinv + b_ref[...], 0.0)
    y = jnp.dot(h, w_ref[...], preferred_element_type=jnp.float32) * dinv
    out_ref[0] = y[:, :128]
    out_ref[1] = y[:, 128:]


def _tail_body(es_ref, y_ref, degp_ref, b_ref, batch_ref, wt1_ref, bt1_ref,
               wt2_ref, bt2_ref, out_ref, sum_ref, cnt_ref):
    i = pl.program_id(0)

    @pl.when(i == 0)
    def _():
        sum_ref[...] = jnp.zeros_like(sum_ref)
        cnt_ref[...] = jnp.zeros_like(cnt_ref)

    dinv = _dinv_of(degp_ref)
    z = jnp.concatenate([es_ref[0] + y_ref[0], es_ref[1] + y_ref[1]], axis=1)
    h = z * dinv + b_ref[...]
    gids = lax.broadcasted_iota(jnp.int32, (NUM_GRAPHS, RBLK), 0)
    onehot = (gids == batch_ref[0]).astype(jnp.float32)
    sum_ref[...] += jnp.dot(onehot, h, preferred_element_type=jnp.float32)
    cnt_ref[...] += jnp.sum(onehot, axis=1, keepdims=True)

    @pl.when(i == NBLK - 1)
    def _():
        pooled = sum_ref[...] / jnp.maximum(cnt_ref[...], 1.0)
        t = jnp.dot(pooled, wt1_ref[...], preferred_element_type=jnp.float32)
        t = jnp.maximum(t + bt1_ref[...], 0.0)
        o = jnp.dot(t, wt2_ref[...], preferred_element_type=jnp.float32)
        out_ref[...] = o + bt2_ref[...]


def kernel(x, edge_index, batch, W_embed, b_embed, W_g0, b_g0, W_g1, b_g1,
           W_g2, b_g2, W_t1, b_t1, W_t2, b_t2):
    f32 = jnp.float32
    x_pad = jnp.pad(x, ((0, NPAD - N), (0, 0)))
    src = edge_index[0]
    dst = edge_index[1]
    pad = jnp.full((EPAD - E,), N, jnp.int32)
    srcp = jnp.concatenate([src, pad])
    dstp = jnp.concatenate([dst, pad])
    src2 = jnp.concatenate([srcp, srcp + NPAD]).reshape(NC * NS, NSEG, SEGC, CHUNK)
    dst3 = dstp.reshape(NS, NSEG, SEGC, CHUNK)
    dstd = dstp.reshape(NC * NS, EPW // CHUNK, CHUNK)
    batch3 = jnp.concatenate(
        [batch, jnp.full((NPAD - N,), NUM_GRAPHS, jnp.int32)]).reshape(NBLK, 1, RBLK)

    be = b_embed.reshape(1, -1)
    bg0 = b_g0.reshape(1, -1)
    bg1 = b_g1.reshape(1, -1)
    bg2 = b_g2.reshape(1, -1)
    bt1 = b_t1.reshape(1, -1)
    bt2 = b_t2.reshape(1, 1)

    degp = _deg_pass(dstd).reshape(NC, NPAD, 16)

    full = lambda shp: pl.BlockSpec(shp, lambda i: tuple(0 for _ in shp))
    rows2 = pl.BlockSpec((2, RBLK, 128), lambda i: (0, i, 0))
    degs = pl.BlockSpec((2, RBLK, 16), lambda i: (0, i, 0))

    y0 = pl.pallas_call(
        _embed_body,
        grid=(NBLK,),
        in_specs=[pl.BlockSpec((RBLK, 128), lambda i: (i, 0)),
                  full((128, 256)), full((1, 256)), full((256, 256)), degs],
        out_specs=rows2,
        out_shape=jax.ShapeDtypeStruct((2, NPAD, 128), f32),
    )(x_pad, W_embed, be, W_g0, degp)

    def mid(es, y, b, w):
        return pl.pallas_call(
            _mid_body,
            grid=(NBLK,),
            in_specs=[rows2, rows2, degs, full((1, 256)), full((256, 256))],
            out_specs=rows2,
            out_shape=jax.ShapeDtypeStruct((2, NPAD, 128), f32),
        )(es, y, degp, b, w)

    es0 = _mp_pass(y0.reshape(NC * NPAD, 128), src2, dst3).reshape(NC, NPAD, 128)
    y1 = mid(es0, y0, bg0, W_g1)
    es1 = _mp_pass(y1.reshape(NC * NPAD, 128), src2, dst3).reshape(NC, NPAD, 128)
    y2 = mid(es1, y1, bg1, W_g2)
    es2 = _mp_pass(y2.reshape(NC * NPAD, 128), src2, dst3).reshape(NC, NPAD, 128)

    out = pl.pallas_call(
        _tail_body,
        grid=(NBLK,),
        in_specs=[rows2, rows2, degs, full((1, 256)),
                  pl.BlockSpec((1, 1, RBLK), lambda i: (i, 0, 0)),
                  full((256, 128)), full((1, 128)), full((128, 1)), full((1, 1))],
        out_specs=pl.BlockSpec((NUM_GRAPHS, 1), lambda i: (0, 0)),
        out_shape=jax.ShapeDtypeStruct((NUM_GRAPHS, 1), f32),
        scratch_shapes=[pltpu.VMEM((NUM_GRAPHS, 256), f32),
                        pltpu.VMEM((NUM_GRAPHS, 1), f32)],
        compiler_params=pltpu.CompilerParams(
            dimension_semantics=("arbitrary",)),
    )(es2, y2, degp, bg2, batch3, W_t1, bt1, W_t2, bt2)
    return out
